# Initial kernel scaffold; baseline (speedup 1.0000x reference)
import functools

import jax
import jax.numpy as jnp
from jax import lax
from jax.experimental import pallas as pl
from jax.experimental.pallas import tpu as pltpu
from jax.experimental.pallas import tpu_sc as plsc

_N = 10000
_D = 256
_E = 160000
_H = 128
_NC = 2
_NS = 16
_B = 128
_CH = 79
_PT = _B * _CH
_EP = _PT * _NS
_NPAD = 10240
_RPT = _NPAD // _NS


def _sc_aggregate(xs, src3, dst3):
    mesh = plsc.VectorSubcoreMesh(core_axis_name="c", subcore_axis_name="s")

    @functools.partial(
        pl.kernel,
        out_type=(
            jax.ShapeDtypeStruct((_NC, _NPAD, _H), jnp.float32),
            jax.ShapeDtypeStruct((_NPAD, 16), jnp.float32),
        ),
        mesh=mesh,
        scratch_types=[
            pltpu.VMEM((_CH, _B), jnp.int32),
            pltpu.VMEM((_CH, _B), jnp.int32),
            pltpu.VMEM((_B, _H), jnp.float32),
            pltpu.VMEM((_B, 16), jnp.float32),
            pltpu.VMEM((_RPT, 16), jnp.float32),
            pltpu.VMEM_SHARED((_NPAD, _H), jnp.float32),
            pltpu.VMEM_SHARED((_NPAD, 16), jnp.float32),
            pltpu.SemaphoreType.DMA,
        ],
    )
    def k(xs_hbm, src_hbm, dst_hbm, sums_hbm, cnt_hbm,
          src_v, dst_v, gbuf, ones_v, zc_v, acc_sh, cnt_sh, sem):
        c = lax.axis_index("c")
        s = lax.axis_index("s")

        pltpu.sync_copy(src_hbm.at[s], src_v)
        pltpu.sync_copy(dst_hbm.at[s], dst_v)

        @pl.when(c == 1)
        def _():
            def row(j, carry):
                def col(k2, carry2):
                    sl = pl.ds(k2 * 16, 16)
                    src_v[j, sl] = src_v[j, sl] + _N
                    return carry2
                return lax.fori_loop(0, _B // 16, col, carry)
            lax.fori_loop(0, _CH, row, 0)

        def fill(j, carry):
            def col(k2, carry2):
                gbuf[j, pl.ds(k2 * 16, 16)] = jnp.zeros((16,), jnp.float32)
                return carry2
            lax.fori_loop(0, _H // 16, col, carry)
            ones_v[j, pl.ds(0, 16)] = jnp.ones((16,), jnp.float32)
            return carry
        lax.fori_loop(0, _B, fill, 0)

        def fillz(j, carry):
            zc_v[j, pl.ds(0, 16)] = jnp.zeros((16,), jnp.float32)
            return carry
        lax.fori_loop(0, _RPT, fillz, 0)

        base = s * _RPT
        def zblk(i, carry):
            pltpu.sync_copy(gbuf, acc_sh.at[pl.ds(base + i * _B, _B)])
            return carry
        lax.fori_loop(0, _RPT // _B, zblk, 0)
        pltpu.sync_copy(zc_v, cnt_sh.at[pl.ds(base, _RPT)])
        plsc.subcore_barrier()

        def chunk(j, carry):
            pltpu.async_copy(xs_hbm.at[src_v.at[j]], gbuf, sem).wait()
            pltpu.sync_copy(gbuf, acc_sh.at[dst_v.at[j]], add=True)

            @pl.when(c == 0)
            def _():
                pltpu.sync_copy(ones_v, cnt_sh.at[dst_v.at[j]], add=True)
            return carry
        lax.fori_loop(0, _CH, chunk, 0)
        plsc.subcore_barrier()

        pltpu.sync_copy(acc_sh.at[pl.ds(base, _RPT)],
                        sums_hbm.at[c, pl.ds(base, _RPT)])

        @pl.when(c == 0)
        def _():
            pltpu.sync_copy(cnt_sh.at[pl.ds(base, _RPT)],
                            cnt_hbm.at[pl.ds(base, _RPT)])

    return k(xs, src3, dst3)


_BN = 1000


def _combine(agg, cnt_blk, xfull, wl, wr, b):
    mean = agg / jnp.maximum(cnt_blk[:, 0:1], 1.0)
    return (lax.dot_general(mean, wl, (((1,), (1,)), ((), ())),
                            preferred_element_type=jnp.float32)
            + lax.dot_general(xfull, wr, (((1,), (1,)), ((), ())),
                              preferred_element_type=jnp.float32)
            + b)


def _l1_body(sums_ref, cnt_ref, x_ref, wl_ref, wr_ref, b_ref, out_ref):
    sm = sums_ref[...]
    agg = jnp.concatenate([sm[0], sm[1]], axis=1)
    res = _combine(agg, cnt_ref[...], x_ref[...], wl_ref[...], wr_ref[...],
                   b_ref[...])
    res = jnp.maximum(res, 0.0)
    out_ref[...] = jnp.stack([res[:, :_H], res[:, _H:]], axis=0)


def _l2_body(sums_ref, cnt_ref, h_ref, wl_ref, wr_ref, b_ref, out_ref):
    sm = sums_ref[...]
    agg = jnp.concatenate([sm[0], sm[1]], axis=1)
    hm = h_ref[...]
    hfull = jnp.concatenate([hm[0], hm[1]], axis=1)
    out_ref[...] = _combine(agg, cnt_ref[...], hfull, wl_ref[...],
                            wr_ref[...], b_ref[...])


_W_SPEC = pl.BlockSpec((_D, _D), lambda i: (0, 0))
_B_SPEC = pl.BlockSpec((1, _D), lambda i: (0, 0))
_SUMS_SPEC = pl.BlockSpec((_NC, _BN, _H), lambda i: (0, i, 0))
_CNT_SPEC = pl.BlockSpec((_BN, 16), lambda i: (i, 0))


def _tc_layer1(sums, cnt, x, wl, wr, b):
    return pl.pallas_call(
        _l1_body,
        grid=(_N // _BN,),
        in_specs=[_SUMS_SPEC, _CNT_SPEC,
                  pl.BlockSpec((_BN, _D), lambda i: (i, 0)),
                  _W_SPEC, _W_SPEC, _B_SPEC],
        out_specs=pl.BlockSpec((_NC, _BN, _H), lambda i: (0, i, 0)),
        out_shape=jax.ShapeDtypeStruct((_NC, _N, _H), jnp.float32),
    )(sums, cnt, x, wl, wr, b)


def _tc_layer2(sums, cnt, hs, wl, wr, b):
    return pl.pallas_call(
        _l2_body,
        grid=(_N // _BN,),
        in_specs=[_SUMS_SPEC, _CNT_SPEC,
                  pl.BlockSpec((_NC, _BN, _H), lambda i: (0, i, 0)),
                  _W_SPEC, _W_SPEC, _B_SPEC],
        out_specs=pl.BlockSpec((_BN, _D), lambda i: (i, 0)),
        out_shape=jax.ShapeDtypeStruct((_N, _D), jnp.float32),
    )(sums, cnt, hs, wl, wr, b)


def kernel(x, edge_index, Wl1, bl1, Wr1, Wl2, bl2, Wr2):
    src = edge_index[0]
    dst = edge_index[1]
    pad = _EP - _E
    src3 = jnp.concatenate([src, jnp.zeros((pad,), jnp.int32)]
                           ).reshape(_NS, _CH, _B)
    dst3 = jnp.concatenate([dst, jnp.full((pad,), _N, jnp.int32)]
                           ).reshape(_NS, _CH, _B)

    xs = jnp.concatenate([x[:, :_H], x[:, _H:]], axis=0)
    sums1, cnt = _sc_aggregate(xs, src3, dst3)
    hs = _tc_layer1(sums1, cnt, x, Wl1, Wr1, bl1.reshape(1, _D))

    xs2 = hs.reshape(_NC * _N, _H)
    sums2, _cnt2 = _sc_aggregate(xs2, src3, dst3)
    out = _tc_layer2(sums2, cnt, hs, Wl2, Wr2, bl2.reshape(1, _D))
    return out


# speedup vs baseline: 3.0028x; 3.0028x over previous
import functools

import jax
import jax.numpy as jnp
from jax import lax
from jax.experimental import pallas as pl
from jax.experimental.pallas import tpu as pltpu
from jax.experimental.pallas import tpu_sc as plsc

_N = 10000
_D = 256
_E = 160000
_Q = 4
_HQ = 64
_NC = 2
_NS = 16
_B = 128
_CH = 79
_PT = _B * _CH
_EP = _PT * _NS
_NPAD = 10240
_RPT = _NPAD // _NS


def _sc_aggregate(xs, src3, dst3, ones8, zcnt8):
    mesh = plsc.VectorSubcoreMesh(core_axis_name="c", subcore_axis_name="s")

    @functools.partial(
        pl.kernel,
        out_type=(
            jax.ShapeDtypeStruct((_Q, _NPAD, _HQ), jnp.float32),
            jax.ShapeDtypeStruct((_NPAD, 8), jnp.float32),
        ),
        mesh=mesh,
        compiler_params=pltpu.CompilerParams(use_tc_tiling_on_sc=False),
        scratch_types=[
            pltpu.VMEM((_CH, _B), jnp.int32),
            pltpu.VMEM((_CH, _B), jnp.int32),
            pltpu.VMEM((_B, _HQ), jnp.float32),
            pltpu.VMEM((_B, 8), jnp.float32),
            pltpu.VMEM_SHARED((_NPAD, _HQ), jnp.float32),
            pltpu.VMEM_SHARED((_NPAD, 8), jnp.float32),
            pltpu.SemaphoreType.DMA,
        ],
    )
    def k(xs_hbm, src_hbm, dst_hbm, ones_hbm, zcnt_hbm, sums_hbm, cnt_hbm,
          src_v, dst_v, gbuf, ones_v, acc_sh, cnt_sh, sem):
        c = lax.axis_index("c")
        s = lax.axis_index("s")

        pltpu.sync_copy(src_hbm.at[s], src_v)
        pltpu.sync_copy(dst_hbm.at[s], dst_v)
        pltpu.sync_copy(ones_hbm, ones_v)

        def add_src_offset(off):
            def row(j, carry):
                def col(k2, carry2):
                    sl = pl.ds(k2 * 16, 16)
                    src_v[j, sl] = src_v[j, sl] + off
                    return carry2
                return lax.fori_loop(0, _B // 16, col, carry)
            lax.fori_loop(0, _CH, row, 0)

        @pl.when(c == 1)
        def _():
            add_src_offset(2 * _N)

        base = s * _RPT
        for p in range(2):
            if p == 1:
                add_src_offset(_N)

            def fill(j, carry):
                def col(k2, carry2):
                    gbuf[j, pl.ds(k2 * 16, 16)] = jnp.zeros((16,),
                                                            jnp.float32)
                    return carry2
                return lax.fori_loop(0, _HQ // 16, col, carry)
            lax.fori_loop(0, _B, fill, 0)

            def zblk(i, carry):
                pltpu.sync_copy(gbuf, acc_sh.at[pl.ds(base + i * _B, _B)])
                return carry
            lax.fori_loop(0, _RPT // _B, zblk, 0)
            if p == 0:
                pltpu.sync_copy(zcnt_hbm, cnt_sh.at[pl.ds(base, _RPT)])
            plsc.subcore_barrier()

            count_here = p == 0

            def chunk(j, carry):
                pltpu.async_copy(xs_hbm.at[src_v.at[j]], gbuf, sem).wait()
                pltpu.sync_copy(gbuf, acc_sh.at[dst_v.at[j]], add=True)
                if count_here:
                    @pl.when(c == 0)
                    def _():
                        pltpu.sync_copy(ones_v, cnt_sh.at[dst_v.at[j]],
                                        add=True)
                return carry
            lax.fori_loop(0, _CH, chunk, 0)
            plsc.subcore_barrier()

            pltpu.sync_copy(acc_sh.at[pl.ds(base, _RPT)],
                            sums_hbm.at[2 * c + p, pl.ds(base, _RPT)])
            if p == 0:
                @pl.when(c == 0)
                def _():
                    pltpu.sync_copy(cnt_sh.at[pl.ds(base, _RPT)],
                                    cnt_hbm.at[pl.ds(base, _RPT)])

    return k(xs, src3, dst3, ones8, zcnt8)


_BN = 1000


def _combine(agg, cnt_blk, xfull, wl, wr, b):
    mean = agg / jnp.maximum(cnt_blk[:, 0:1], 1.0)
    return (lax.dot_general(mean, wl, (((1,), (1,)), ((), ())),
                            preferred_element_type=jnp.float32)
            + lax.dot_general(xfull, wr, (((1,), (1,)), ((), ())),
                              preferred_element_type=jnp.float32)
            + b)


def _l1_body(sums_ref, cnt_ref, x_ref, wl_ref, wr_ref, b_ref, out_ref):
    sm = sums_ref[...]
    agg = jnp.concatenate([sm[0], sm[1], sm[2], sm[3]], axis=1)
    res = _combine(agg, cnt_ref[...], x_ref[...], wl_ref[...], wr_ref[...],
                   b_ref[...])
    res = jnp.maximum(res, 0.0)
    out_ref[...] = jnp.stack(
        [res[:, q * _HQ:(q + 1) * _HQ] for q in range(_Q)], axis=0)


def _l2_body(sums_ref, cnt_ref, h_ref, wl_ref, wr_ref, b_ref, out_ref):
    sm = sums_ref[...]
    agg = jnp.concatenate([sm[0], sm[1], sm[2], sm[3]], axis=1)
    hm = h_ref[...]
    hfull = jnp.concatenate([hm[0], hm[1], hm[2], hm[3]], axis=1)
    out_ref[...] = _combine(agg, cnt_ref[...], hfull, wl_ref[...],
                            wr_ref[...], b_ref[...])


_W_SPEC = pl.BlockSpec((_D, _D), lambda i: (0, 0))
_B_SPEC = pl.BlockSpec((1, _D), lambda i: (0, 0))
_SUMS_SPEC = pl.BlockSpec((_Q, _BN, _HQ), lambda i: (0, i, 0))
_CNT_SPEC = pl.BlockSpec((_BN, 8), lambda i: (i, 0))
_HS_SPEC = pl.BlockSpec((_Q, _BN, _HQ), lambda i: (0, i, 0))


def _tc_layer1(sums, cnt, x, wl, wr, b):
    return pl.pallas_call(
        _l1_body,
        grid=(_N // _BN,),
        in_specs=[_SUMS_SPEC, _CNT_SPEC,
                  pl.BlockSpec((_BN, _D), lambda i: (i, 0)),
                  _W_SPEC, _W_SPEC, _B_SPEC],
        out_specs=_HS_SPEC,
        out_shape=jax.ShapeDtypeStruct((_Q, _N, _HQ), jnp.float32),
    )(sums, cnt, x, wl, wr, b)


def _tc_layer2(sums, cnt, hs, wl, wr, b):
    return pl.pallas_call(
        _l2_body,
        grid=(_N // _BN,),
        in_specs=[_SUMS_SPEC, _CNT_SPEC, _HS_SPEC,
                  _W_SPEC, _W_SPEC, _B_SPEC],
        out_specs=pl.BlockSpec((_BN, _D), lambda i: (i, 0)),
        out_shape=jax.ShapeDtypeStruct((_N, _D), jnp.float32),
    )(sums, cnt, hs, wl, wr, b)


def kernel(x, edge_index, Wl1, bl1, Wr1, Wl2, bl2, Wr2):
    src = edge_index[0]
    dst = edge_index[1]
    pad = _EP - _E
    src3 = jnp.concatenate([src, jnp.zeros((pad,), jnp.int32)]
                           ).reshape(_NS, _CH, _B)
    dst3 = jnp.concatenate([dst, jnp.full((pad,), _N, jnp.int32)]
                           ).reshape(_NS, _CH, _B)

    ones8 = jnp.ones((_B, 8), jnp.float32)
    zcnt8 = jnp.zeros((_RPT, 8), jnp.float32)

    xs = jnp.concatenate([x[:, q * _HQ:(q + 1) * _HQ] for q in range(_Q)],
                         axis=0)
    sums1, cnt = _sc_aggregate(xs, src3, dst3, ones8, zcnt8)
    hs = _tc_layer1(sums1, cnt, x, Wl1, Wr1, bl1.reshape(1, _D))

    xs2 = hs.reshape(_Q * _N, _HQ)
    sums2, _cnt2 = _sc_aggregate(xs2, src3, dst3, ones8, zcnt8)
    out = _tc_layer2(sums2, cnt, hs, Wl2, Wr2, bl2.reshape(1, _D))
    return out


# speedup vs baseline: 4.3187x; 1.4382x over previous
import functools

import jax
import jax.numpy as jnp
from jax import lax
from jax.experimental import pallas as pl
from jax.experimental.pallas import tpu as pltpu
from jax.experimental.pallas import tpu_sc as plsc

_N = 10000
_D = 256
_E = 160000
_Q = 2
_HQ = _D // _Q
_PASSES = _Q // 2
_NC = 2
_NS = 16
_B = 64
_CH = 158
_PT = _B * _CH
_EP = _PT * _NS
_NPAD = 10240
_RPT = _NPAD // _NS


def _sc_aggregate(xs, src3, dst3, ones8, zcnt8):
    mesh = plsc.VectorSubcoreMesh(core_axis_name="c", subcore_axis_name="s")

    @functools.partial(
        pl.kernel,
        out_type=(
            jax.ShapeDtypeStruct((_Q, _NPAD, _HQ), jnp.float32),
            jax.ShapeDtypeStruct((_NPAD, 8), jnp.float32),
        ),
        mesh=mesh,
        compiler_params=pltpu.CompilerParams(use_tc_tiling_on_sc=False),
        scratch_types=[
            pltpu.VMEM((_CH, _B), jnp.int32),
            pltpu.VMEM((_CH, _B), jnp.int32),
            pltpu.VMEM((2, _B, _HQ), jnp.float32),
            pltpu.VMEM((_B, 8), jnp.float32),
            pltpu.VMEM_SHARED((_NPAD, _HQ), jnp.float32),
            pltpu.VMEM_SHARED((_NPAD, 8), jnp.float32),
            pltpu.SemaphoreType.DMA,
        ],
    )
    def k(xs_hbm, src_hbm, dst_hbm, ones_hbm, zcnt_hbm, sums_hbm, cnt_hbm,
          src_v, dst_v, gb2, ones_v, acc_sh, cnt_sh, sem):
        c = lax.axis_index("c")
        s = lax.axis_index("s")

        pltpu.sync_copy(src_hbm.at[s], src_v)
        pltpu.sync_copy(dst_hbm.at[s], dst_v)
        pltpu.sync_copy(ones_hbm, ones_v)

        def add_src_offset(off):
            def row(j, carry):
                def col(k2, carry2):
                    sl = pl.ds(k2 * 16, 16)
                    src_v[j, sl] = src_v[j, sl] + off
                    return carry2
                return lax.fori_loop(0, _B // 16, col, carry)
            lax.fori_loop(0, _CH, row, 0)

        @pl.when(c == 1)
        def _():
            add_src_offset(_PASSES * _N)

        base = s * _RPT
        for p in range(_PASSES):
            if p == 1:
                add_src_offset(_N)

            def fill(j, carry):
                def col(k2, carry2):
                    gb2[0, j, pl.ds(k2 * 16, 16)] = jnp.zeros((16,),
                                                              jnp.float32)
                    return carry2
                return lax.fori_loop(0, _HQ // 16, col, carry)
            lax.fori_loop(0, _B, fill, 0)

            def zblk(i, carry):
                pltpu.sync_copy(gb2.at[0],
                                acc_sh.at[pl.ds(base + i * _B, _B)])
                return carry
            lax.fori_loop(0, _RPT // _B, zblk, 0)
            if p == 0:
                pltpu.sync_copy(zcnt_hbm, cnt_sh.at[pl.ds(base, _RPT)])
            plsc.subcore_barrier()

            count_here = p == 0
            pltpu.async_copy(xs_hbm.at[src_v.at[0]], gb2.at[0], sem)

            def chunk(j, carry):
                par = lax.rem(j, 2)
                pltpu.make_async_copy(xs_hbm.at[src_v.at[j]], gb2.at[par],
                                      sem).wait()

                @pl.when(j + 1 < _CH)
                def _():
                    pltpu.async_copy(xs_hbm.at[src_v.at[j + 1]],
                                     gb2.at[1 - par], sem)
                pltpu.sync_copy(gb2.at[par], acc_sh.at[dst_v.at[j]],
                                add=True)
                if count_here:
                    @pl.when(c == 0)
                    def _():
                        pltpu.sync_copy(ones_v, cnt_sh.at[dst_v.at[j]],
                                        add=True)
                return carry
            lax.fori_loop(0, _CH, chunk, 0)
            plsc.subcore_barrier()

            pltpu.sync_copy(acc_sh.at[pl.ds(base, _RPT)],
                            sums_hbm.at[_PASSES * c + p, pl.ds(base, _RPT)])
            if p == 0:
                @pl.when(c == 0)
                def _():
                    pltpu.sync_copy(cnt_sh.at[pl.ds(base, _RPT)],
                                    cnt_hbm.at[pl.ds(base, _RPT)])

    return k(xs, src3, dst3, ones8, zcnt8)


_BN = 1000


def _combine(agg, cnt_blk, xfull, wl, wr, b):
    mean = agg / jnp.maximum(cnt_blk[:, 0:1], 1.0)
    return (lax.dot_general(mean, wl, (((1,), (1,)), ((), ())),
                            preferred_element_type=jnp.float32)
            + lax.dot_general(xfull, wr, (((1,), (1,)), ((), ())),
                              preferred_element_type=jnp.float32)
            + b)


def _l1_body(sums_ref, cnt_ref, x_ref, wl_ref, wr_ref, b_ref, out_ref):
    sm = sums_ref[...]
    agg = jnp.concatenate([sm[q] for q in range(_Q)], axis=1)
    res = _combine(agg, cnt_ref[...], x_ref[...], wl_ref[...],
                   wr_ref[...], b_ref[...])
    res = jnp.maximum(res, 0.0)
    out_ref[...] = jnp.stack(
        [res[:, q * _HQ:(q + 1) * _HQ] for q in range(_Q)], axis=0)


def _l2_body(sums_ref, cnt_ref, h_ref, wl_ref, wr_ref, b_ref, out_ref):
    sm = sums_ref[...]
    agg = jnp.concatenate([sm[q] for q in range(_Q)], axis=1)
    hm = h_ref[...]
    hfull = jnp.concatenate([hm[q] for q in range(_Q)], axis=1)
    out_ref[...] = _combine(agg, cnt_ref[...], hfull, wl_ref[...],
                            wr_ref[...], b_ref[...])


_W_SPEC = pl.BlockSpec((_D, _D), lambda i: (0, 0))
_B_SPEC = pl.BlockSpec((1, _D), lambda i: (0, 0))
_SUMS_SPEC = pl.BlockSpec((_Q, _BN, _HQ), lambda i: (0, i, 0))
_CNT_SPEC = pl.BlockSpec((_BN, 8), lambda i: (i, 0))
_HS_SPEC = pl.BlockSpec((_Q, _BN, _HQ), lambda i: (0, i, 0))


def _tc_layer1(sums, cnt, x, wl, wr, b):
    return pl.pallas_call(
        _l1_body,
        grid=(_N // _BN,),
        in_specs=[_SUMS_SPEC, _CNT_SPEC,
                  pl.BlockSpec((_BN, _D), lambda i: (i, 0)),
                  _W_SPEC, _W_SPEC, _B_SPEC],
        out_specs=_HS_SPEC,
        out_shape=jax.ShapeDtypeStruct((_Q, _N, _HQ), jnp.float32),
    )(sums, cnt, x, wl, wr, b)


def _tc_layer2(sums, cnt, hs, wl, wr, b):
    return pl.pallas_call(
        _l2_body,
        grid=(_N // _BN,),
        in_specs=[_SUMS_SPEC, _CNT_SPEC, _HS_SPEC,
                  _W_SPEC, _W_SPEC, _B_SPEC],
        out_specs=pl.BlockSpec((_BN, _D), lambda i: (i, 0)),
        out_shape=jax.ShapeDtypeStruct((_N, _D), jnp.float32),
    )(sums, cnt, hs, wl, wr, b)


def kernel(x, edge_index, Wl1, bl1, Wr1, Wl2, bl2, Wr2):
    src = edge_index[0]
    dst = edge_index[1]
    pad = _EP - _E
    src3 = jnp.concatenate([src, jnp.zeros((pad,), jnp.int32)]
                           ).reshape(_NS, _CH, _B)
    dst3 = jnp.concatenate([dst, jnp.full((pad,), _N, jnp.int32)]
                           ).reshape(_NS, _CH, _B)

    ones8 = jnp.ones((_B, 8), jnp.float32)
    zcnt8 = jnp.zeros((_RPT, 8), jnp.float32)

    xs = jnp.concatenate([x[:, q * _HQ:(q + 1) * _HQ] for q in range(_Q)],
                         axis=0)
    sums1, cnt = _sc_aggregate(xs, src3, dst3, ones8, zcnt8)
    hs = _tc_layer1(sums1, cnt, x, Wl1, Wr1, bl1.reshape(1, _D))

    xs2 = hs.reshape(_Q * _N, _HQ)
    sums2, _cnt2 = _sc_aggregate(xs2, src3, dst3, ones8, zcnt8)
    out = _tc_layer2(sums2, cnt, hs, Wl2, Wr2, bl2.reshape(1, _D))
    return out
